# Initial kernel scaffold; baseline (speedup 1.0000x reference)
import jax
import jax.numpy as jnp
from jax.experimental import pallas as pl


def kernel(x, edge_index, W1, b1, W2, b2, Wm1, bm1, Wm2, bm2):
    raise NotImplementedError("write your pallas kernel here")


# speedup vs baseline: 4.5655x; 4.5655x over previous
import functools

import jax
import jax.numpy as jnp
from jax import lax
from jax.experimental import pallas as pl
from jax.experimental.pallas import tpu as pltpu
from jax.experimental.pallas import tpu_sc as plsc

NC = 2
NS = 16
L = 16

N_PAD = 10240
ACC_ROWS = N_PAD + 128
EB = 128


def _sc_mesh():
    return plsc.VectorSubcoreMesh(core_axis_name="c", subcore_axis_name="s")


DW = 128


def _deg_body(nb, dst_hbm, out_hbm, dstv, ones_rows, acc, ss):
    c = lax.axis_index("c")
    s = lax.axis_index("s")
    zrows = ACC_ROWS // NS
    orows = N_PAD // NS

    one = jnp.full((L,), 1.0, jnp.float32)
    zero = jnp.zeros((L,), jnp.float32)

    pltpu.sync_copy(dst_hbm.at[c, s], dstv)

    @pl.loop(0, EB)
    def _zz(r):
        for m in range(8):
            ones_rows[r, pl.ds(m * L, L)] = zero

    for z in range(zrows // EB):
        pltpu.sync_copy(ones_rows, acc.at[pl.ds(s * zrows + z * EB, EB)])
    rem = zrows % EB
    if rem:
        pltpu.sync_copy(ones_rows.at[pl.ds(0, rem)],
                        acc.at[pl.ds(s * zrows + zrows - rem, rem)])

    @pl.loop(0, EB)
    def _o(r):
        for m in range(8):
            ones_rows[r, pl.ds(m * L, L)] = one

    plsc.subcore_barrier()

    @pl.loop(0, nb)
    def _batches(j):
        pltpu.sync_copy(ones_rows, acc.at[dstv.at[j]], add=True)

    plsc.subcore_barrier()
    pltpu.sync_copy(acc.at[pl.ds(s * orows, orows)],
                    out_hbm.at[c, pl.ds(s * orows, orows)])


def _deg_call(dstp):
    ep = dstp.shape[0]
    nb = ep // (NC * NS * EB)
    dst4 = dstp.reshape(NC, NS, nb, EB)
    f = pl.kernel(
        functools.partial(_deg_body, nb),
        out_type=jax.ShapeDtypeStruct((NC, N_PAD, DW), jnp.float32),
        mesh=_sc_mesh(),
        scratch_types=[
            pltpu.VMEM((nb, EB), jnp.int32),
            pltpu.VMEM((EB, DW), jnp.float32),
            pltpu.VMEM_SHARED((ACC_ROWS, DW), jnp.float32),
            pltpu.SemaphoreType.DMA,
        ],
    )
    return f(dst4)


def _agg_body(cdiv, nb, table, src2, dst2, out_hbm,
              srcv, dstv, gi, rows, acc, gs):
    c = lax.axis_index("c")
    s = lax.axis_index("s")
    zrows = ACC_ROWS // NS
    orows = N_PAD // NS

    pltpu.sync_copy(src2.at[s], srcv)
    pltpu.sync_copy(dst2.at[s], dstv)

    zero = jnp.zeros((L,), jnp.float32)
    mulc = NC * cdiv

    for ci in range(cdiv):
        chunk = c * cdiv + ci

        @pl.loop(0, EB)
        def _zr(r):
            for m in range(8):
                rows[r, pl.ds(m * L, L)] = zero

        for z in range(zrows // EB):
            pltpu.sync_copy(rows, acc.at[pl.ds(s * zrows + z * EB, EB)])
        rem = zrows % EB
        if rem:
            pltpu.sync_copy(rows.at[pl.ds(0, rem)],
                            acc.at[pl.ds(s * zrows + zrows - rem, rem)])
        plsc.subcore_barrier()

        @pl.loop(0, nb)
        def _batches(j):
            for k in range(8):
                sl = pl.ds(k * L, L)
                gi[sl] = srcv[j, sl] * mulc + chunk
            pltpu.async_copy(table.at[gi], rows, gs).wait()
            pltpu.sync_copy(rows, acc.at[dstv.at[j]], add=True)

        plsc.subcore_barrier()
        pltpu.sync_copy(acc.at[pl.ds(s * orows, orows)],
                        out_hbm.at[pl.ds(s * orows, orows), chunk])
        plsc.subcore_barrier()


def _agg_call(table, src2, dst2, n_chunks):
    cdiv = n_chunks // NC
    nb = src2.shape[1]
    f = pl.kernel(
        functools.partial(_agg_body, cdiv, nb),
        out_type=jax.ShapeDtypeStruct((N_PAD, n_chunks, 128), jnp.float32),
        mesh=_sc_mesh(),
        scratch_types=[
            pltpu.VMEM((nb, EB), jnp.int32),
            pltpu.VMEM((nb, EB), jnp.int32),
            pltpu.VMEM((EB,), jnp.int32),
            pltpu.VMEM((EB, 128), jnp.float32),
            pltpu.VMEM_SHARED((ACC_ROWS, 128), jnp.float32),
            pltpu.SemaphoreType.DMA,
        ],
    )
    return f(table, src2, dst2)


def _dinv_body(deg_ref, out_ref):
    d = deg_ref[0] + deg_ref[1]
    out_ref[...] = jnp.where(d > 0, lax.rsqrt(d), 0.0)


def _dinv_call(deg2):
    return pl.pallas_call(
        _dinv_body,
        out_shape=jax.ShapeDtypeStruct((N_PAD, DW), jnp.float32),
    )(deg2)


def _scale_body(x_ref, s_ref, out_ref):
    out_ref[...] = x_ref[...] * s_ref[...]


def _scale_call(x, scale_col):
    bm = 512
    n, d = x.shape
    return pl.pallas_call(
        _scale_body,
        grid=(n // bm,),
        in_specs=[
            pl.BlockSpec((bm, d), lambda i: (i, 0)),
            pl.BlockSpec((bm, 1), lambda i: (i, 0)),
        ],
        out_specs=pl.BlockSpec((bm, d), lambda i: (i, 0)),
        out_shape=jax.ShapeDtypeStruct((n, d), jnp.float32),
    )(x, scale_col)


def _mm_body(has_scale, has_preb, pre_relu, has_postb, post_relu, *refs):
    refs = list(refs)
    x_ref = refs.pop(0)
    w_ref = refs.pop(0)
    s_ref = refs.pop(0) if has_scale else None
    pb_ref = refs.pop(0) if has_preb else None
    qb_ref = refs.pop(0) if has_postb else None
    out_ref = refs.pop(0)

    xb = x_ref[...]
    if has_scale:
        xb = xb * s_ref[...]
    if has_preb:
        xb = xb + pb_ref[...]
    if pre_relu:
        xb = jnp.maximum(xb, 0.0)
    acc = lax.dot_general(xb, w_ref[...], (((1,), (1,)), ((), ())),
                          preferred_element_type=jnp.float32)
    if has_postb:
        acc = acc + qb_ref[...]
    if post_relu:
        acc = jnp.maximum(acc, 0.0)
    out_ref[...] = acc


def _mm_call(x, w, scale_col=None, pre_bias=None, pre_relu=False,
             post_bias=None, post_relu=False):
    n, k = x.shape
    dout = w.shape[0]
    bm = 512
    ops = [x, w]
    specs = [pl.BlockSpec((bm, k), lambda i: (i, 0)),
             pl.BlockSpec((dout, k), lambda i: (0, 0))]
    if scale_col is not None:
        ops.append(scale_col)
        specs.append(pl.BlockSpec((bm, 1), lambda i: (i, 0)))
    if pre_bias is not None:
        ops.append(pre_bias.reshape(1, k))
        specs.append(pl.BlockSpec((1, k), lambda i: (0, 0)))
    if post_bias is not None:
        ops.append(post_bias.reshape(1, dout))
        specs.append(pl.BlockSpec((1, dout), lambda i: (0, 0)))
    body = functools.partial(_mm_body, scale_col is not None,
                             pre_bias is not None, pre_relu,
                             post_bias is not None, post_relu)
    return pl.pallas_call(
        body,
        grid=(n // bm,),
        in_specs=specs,
        out_specs=pl.BlockSpec((bm, dout), lambda i: (i, 0)),
        out_shape=jax.ShapeDtypeStruct((n, dout), jnp.float32),
    )(*ops)


def kernel(x, edge_index, W1, b1, W2, b2, Wm1, bm1, Wm2, bm2):
    n, d_in = x.shape
    e = edge_index.shape[1]
    ep = NS * N_PAD

    xp = jnp.pad(x, ((0, N_PAD - n), (0, 0)))
    src = edge_index[0]
    dst = edge_index[1]
    srcp = jnp.concatenate([src, jnp.zeros((ep - e,), jnp.int32)])
    dstp = jnp.concatenate([dst, jnp.full((ep - e,), N_PAD, jnp.int32)])
    src2 = srcp.reshape(NS, N_PAD // EB, EB)
    dst2 = dstp.reshape(NS, N_PAD // EB, EB)

    deg2 = _deg_call(dstp)
    dinv_col = _dinv_call(deg2)[:, :1]

    xs = _scale_call(xp, dinv_col)
    g1 = _agg_call(xs.reshape(N_PAD * 2, 128), src2, dst2, 2)
    h1 = _mm_call(g1.reshape(N_PAD, d_in), W1, scale_col=dinv_col,
                  post_bias=b1, post_relu=True)
    t2 = _mm_call(h1, W2, scale_col=dinv_col)
    g2 = _agg_call(t2.reshape(N_PAD * 4, 128), src2, dst2, 4)
    m = _mm_call(g2.reshape(N_PAD, W2.shape[0]), Wm1, scale_col=dinv_col,
                 pre_bias=b2, pre_relu=True, post_bias=bm1, post_relu=True)
    out = _mm_call(m, Wm2, post_bias=bm2)
    return out[:n]


# speedup vs baseline: 5.3729x; 1.1769x over previous
import functools

import jax
import jax.numpy as jnp
from jax import lax
from jax.experimental import pallas as pl
from jax.experimental.pallas import tpu as pltpu
from jax.experimental.pallas import tpu_sc as plsc

NC = 2
NS = 16
L = 16

N_PAD = 10240
ACC_ROWS = N_PAD + 128
EB = 128


def _sc_mesh():
    return plsc.VectorSubcoreMesh(core_axis_name="c", subcore_axis_name="s")


DW = 128


def _deg_body(nb, dst_hbm, out_hbm, dstv, ones_rows, acc, ss):
    c = lax.axis_index("c")
    s = lax.axis_index("s")
    zrows = ACC_ROWS // NS
    orows = N_PAD // NS

    one = jnp.full((L,), 1.0, jnp.float32)
    zero = jnp.zeros((L,), jnp.float32)

    pltpu.sync_copy(dst_hbm.at[c, s], dstv)

    @pl.loop(0, EB)
    def _zz(r):
        for m in range(8):
            ones_rows[r, pl.ds(m * L, L)] = zero

    for z in range(zrows // EB):
        pltpu.sync_copy(ones_rows, acc.at[pl.ds(s * zrows + z * EB, EB)])
    rem = zrows % EB
    if rem:
        pltpu.sync_copy(ones_rows.at[pl.ds(0, rem)],
                        acc.at[pl.ds(s * zrows + zrows - rem, rem)])

    @pl.loop(0, EB)
    def _o(r):
        for m in range(8):
            ones_rows[r, pl.ds(m * L, L)] = one

    plsc.subcore_barrier()

    @pl.loop(0, nb)
    def _batches(j):
        pltpu.sync_copy(ones_rows, acc.at[dstv.at[j]], add=True)

    plsc.subcore_barrier()
    pltpu.sync_copy(acc.at[pl.ds(s * orows, orows)],
                    out_hbm.at[c, pl.ds(s * orows, orows)])


def _deg_call(dstp):
    ep = dstp.shape[0]
    nb = ep // (NC * NS * EB)
    dst4 = dstp.reshape(NC, NS, nb, EB)
    f = pl.kernel(
        functools.partial(_deg_body, nb),
        out_type=jax.ShapeDtypeStruct((NC, N_PAD, DW), jnp.float32),
        mesh=_sc_mesh(),
        scratch_types=[
            pltpu.VMEM((nb, EB), jnp.int32),
            pltpu.VMEM((EB, DW), jnp.float32),
            pltpu.VMEM_SHARED((ACC_ROWS, DW), jnp.float32),
            pltpu.SemaphoreType.DMA,
        ],
    )
    return f(dst4)


def _agg_body(cdiv, nb, table, src2, dst2, out_hbm,
              srcv, dv, gi0, gi1, rb0, rb1, acc, gs0, gs1, ds0, ds1):
    c = lax.axis_index("c")
    s = lax.axis_index("s")
    zrows = ACC_ROWS // NS
    orows = N_PAD // NS

    pltpu.sync_copy(src2.at[s], srcv)

    zero = jnp.zeros((L,), jnp.float32)
    mulc = NC * cdiv
    gis = (gi0, gi1)
    rbs = (rb0, rb1)
    gss = (gs0, gs1)
    dss = (ds0, ds1)

    def compute_gi(buf, j, chunk):
        for k in range(8):
            sl = pl.ds(k * L, L)
            buf[sl] = srcv[j, sl] * mulc + chunk

    for ci in range(cdiv):
        chunk = c * cdiv + ci

        @pl.loop(0, EB)
        def _zr(r):
            for m in range(8):
                rb0[r, pl.ds(m * L, L)] = zero

        for z in range(zrows // EB):
            pltpu.sync_copy(rb0, acc.at[pl.ds(s * zrows + z * EB, EB)])
        rem = zrows % EB
        if rem:
            pltpu.sync_copy(rb0.at[pl.ds(0, rem)],
                            acc.at[pl.ds(s * zrows + zrows - rem, rem)])
        plsc.subcore_barrier()

        compute_gi(gis[0], 0, chunk)
        pltpu.async_copy(table.at[gis[0]], rbs[0], gss[0])
        pltpu.async_copy(dst2.at[s, 0], dv.at[0], dss[0])

        @pl.loop(0, nb, step=2)
        def _batches(jj):
            for b in range(2):
                j = jj + b
                nxt = 1 - b

                @pl.when(j + 1 < nb)
                def _prefetch():
                    compute_gi(gis[nxt], j + 1, chunk)
                    pltpu.async_copy(table.at[gis[nxt]], rbs[nxt], gss[nxt])
                    pltpu.async_copy(dst2.at[s, j + 1], dv.at[nxt], dss[nxt])

                pltpu.make_async_copy(table.at[gis[b]], rbs[b], gss[b]).wait()
                pltpu.make_async_copy(dst2.at[s, j], dv.at[b], dss[b]).wait()
                pltpu.sync_copy(rbs[b], acc.at[dv.at[b]], add=True)

        plsc.subcore_barrier()
        pltpu.sync_copy(acc.at[pl.ds(s * orows, orows)],
                        out_hbm.at[pl.ds(s * orows, orows), chunk])
        plsc.subcore_barrier()


def _agg_call(table, src2, dst2, n_chunks):
    cdiv = n_chunks // NC
    nb = src2.shape[1]
    f = pl.kernel(
        functools.partial(_agg_body, cdiv, nb),
        out_type=jax.ShapeDtypeStruct((N_PAD, n_chunks, 128), jnp.float32),
        mesh=_sc_mesh(),
        scratch_types=[
            pltpu.VMEM((nb, EB), jnp.int32),
            pltpu.VMEM((2, EB), jnp.int32),
            pltpu.VMEM((EB,), jnp.int32),
            pltpu.VMEM((EB,), jnp.int32),
            pltpu.VMEM((EB, 128), jnp.float32),
            pltpu.VMEM((EB, 128), jnp.float32),
            pltpu.VMEM_SHARED((ACC_ROWS, 128), jnp.float32),
            pltpu.SemaphoreType.DMA,
            pltpu.SemaphoreType.DMA,
            pltpu.SemaphoreType.DMA,
            pltpu.SemaphoreType.DMA,
        ],
    )
    return f(table, src2, dst2)


def _dinv_body(deg_ref, out_ref):
    d = deg_ref[0] + deg_ref[1]
    out_ref[...] = jnp.where(d > 0, lax.rsqrt(d), 0.0)


def _dinv_call(deg2):
    return pl.pallas_call(
        _dinv_body,
        out_shape=jax.ShapeDtypeStruct((N_PAD, DW), jnp.float32),
    )(deg2)


def _scale_body(x_ref, s_ref, out_ref):
    out_ref[...] = x_ref[...] * s_ref[...]


def _scale_call(x, scale_col):
    bm = 512
    n, d = x.shape
    return pl.pallas_call(
        _scale_body,
        grid=(n // bm,),
        in_specs=[
            pl.BlockSpec((bm, d), lambda i: (i, 0)),
            pl.BlockSpec((bm, 1), lambda i: (i, 0)),
        ],
        out_specs=pl.BlockSpec((bm, d), lambda i: (i, 0)),
        out_shape=jax.ShapeDtypeStruct((n, d), jnp.float32),
    )(x, scale_col)


def _mm_body(has_scale, has_preb, pre_relu, has_postb, post_relu, *refs):
    refs = list(refs)
    x_ref = refs.pop(0)
    w_ref = refs.pop(0)
    s_ref = refs.pop(0) if has_scale else None
    pb_ref = refs.pop(0) if has_preb else None
    qb_ref = refs.pop(0) if has_postb else None
    out_ref = refs.pop(0)

    xb = x_ref[...]
    if has_scale:
        xb = xb * s_ref[...]
    if has_preb:
        xb = xb + pb_ref[...]
    if pre_relu:
        xb = jnp.maximum(xb, 0.0)
    acc = lax.dot_general(xb, w_ref[...], (((1,), (1,)), ((), ())),
                          preferred_element_type=jnp.float32)
    if has_postb:
        acc = acc + qb_ref[...]
    if post_relu:
        acc = jnp.maximum(acc, 0.0)
    out_ref[...] = acc


def _mm_call(x, w, scale_col=None, pre_bias=None, pre_relu=False,
             post_bias=None, post_relu=False):
    n, k = x.shape
    dout = w.shape[0]
    bm = 512
    ops = [x, w]
    specs = [pl.BlockSpec((bm, k), lambda i: (i, 0)),
             pl.BlockSpec((dout, k), lambda i: (0, 0))]
    if scale_col is not None:
        ops.append(scale_col)
        specs.append(pl.BlockSpec((bm, 1), lambda i: (i, 0)))
    if pre_bias is not None:
        ops.append(pre_bias.reshape(1, k))
        specs.append(pl.BlockSpec((1, k), lambda i: (0, 0)))
    if post_bias is not None:
        ops.append(post_bias.reshape(1, dout))
        specs.append(pl.BlockSpec((1, dout), lambda i: (0, 0)))
    body = functools.partial(_mm_body, scale_col is not None,
                             pre_bias is not None, pre_relu,
                             post_bias is not None, post_relu)
    return pl.pallas_call(
        body,
        grid=(n // bm,),
        in_specs=specs,
        out_specs=pl.BlockSpec((bm, dout), lambda i: (i, 0)),
        out_shape=jax.ShapeDtypeStruct((n, dout), jnp.float32),
    )(*ops)


def kernel(x, edge_index, W1, b1, W2, b2, Wm1, bm1, Wm2, bm2):
    n, d_in = x.shape
    e = edge_index.shape[1]
    ep = NS * N_PAD

    xp = jnp.pad(x, ((0, N_PAD - n), (0, 0)))
    src = edge_index[0]
    dst = edge_index[1]
    srcp = jnp.concatenate([src, jnp.zeros((ep - e,), jnp.int32)])
    dstp = jnp.concatenate([dst, jnp.full((ep - e,), N_PAD, jnp.int32)])
    src2 = srcp.reshape(NS, N_PAD // EB, EB)
    dst2 = dstp.reshape(NS, N_PAD // EB, EB)

    deg2 = _deg_call(dstp)
    dinv_col = _dinv_call(deg2)[:, :1]

    xs = _scale_call(xp, dinv_col)
    g1 = _agg_call(xs.reshape(N_PAD * 2, 128), src2, dst2, 2)
    h1 = _mm_call(g1.reshape(N_PAD, d_in), W1, scale_col=dinv_col,
                  post_bias=b1, post_relu=True)
    t2 = _mm_call(h1, W2, scale_col=dinv_col)
    g2 = _agg_call(t2.reshape(N_PAD * 4, 128), src2, dst2, 4)
    m = _mm_call(g2.reshape(N_PAD, W2.shape[0]), Wm1, scale_col=dinv_col,
                 pre_bias=b2, pre_relu=True, post_bias=bm1, post_relu=True)
    out = _mm_call(m, Wm2, post_bias=bm2)
    return out[:n]
